# baseline (speedup 1.0000x reference)
import functools
import math

import jax
import jax.numpy as jnp
import numpy as np
from jax import lax
from jax.experimental import pallas as pl
from jax.experimental.pallas import tpu as pltpu
from jax.experimental.pallas import tpu_sc as plsc

D_MODEL = 1024
MAX_LEN = 5000
PE_DIM = D_MODEL // 4


def _pe_table() -> np.ndarray:
    position = np.arange(MAX_LEN, dtype=np.float32)[:, None]
    div_term = np.exp(
        np.arange(0, PE_DIM, 2).astype(np.float32) * (-math.log(10000.0) / PE_DIM)
    )
    pe = np.zeros((MAX_LEN, PE_DIM), dtype=np.float32)
    pe[:, 0::2] = np.sin(position * div_term)
    pe[:, 1::2] = np.cos(position * div_term)
    return pe


_PE = _pe_table()

_CHUNK = 64


def _sc_gather(pe, idx, n_rows):
    info = plsc.get_sparse_core_info()
    nc, ns = info.num_cores, info.num_subcores
    nw = nc * ns
    rows_per_w = n_rows // nw
    assert rows_per_w * nw == n_rows and rows_per_w % _CHUNK == 0
    n_chunks = rows_per_w // _CHUNK

    mesh = plsc.VectorSubcoreMesh(core_axis_name="c", subcore_axis_name="s")

    @functools.partial(
        pl.kernel,
        mesh=mesh,
        out_type=jax.ShapeDtypeStruct((n_rows, PE_DIM), jnp.float32),
        scratch_types=[
            pltpu.VMEM((_CHUNK,), jnp.int32),
            pltpu.VMEM((_CHUNK, PE_DIM), jnp.float32),
            pltpu.SemaphoreType.DMA,
        ],
    )
    def k(pe_hbm, idx_hbm, out_hbm, idx_v, rows_v, sg):
        wid = lax.axis_index("s") * nc + lax.axis_index("c")
        w_base = wid * rows_per_w

        def body(u, carry):
            base = w_base + u * _CHUNK
            pltpu.sync_copy(idx_hbm.at[pl.ds(base, _CHUNK)], idx_v)
            pltpu.async_copy(pe_hbm.at[idx_v], rows_v, sg).wait()
            pltpu.sync_copy(rows_v, out_hbm.at[pl.ds(base, _CHUNK)])
            return carry

        lax.fori_loop(0, n_chunks, body, 0)

    return k(pe, idx)


def _tc_add(seq, emb128, s_pad, cb):
    b, s, d = seq.shape
    n_tau = s_pad // (8 * cb)
    spec_sd = pl.BlockSpec((1, 8 * cb, d), lambda i, t: (i, t, 0))

    def body(seq_ref, emb_ref, out_ref, embout_ref):
        e = emb_ref[...].reshape(1, 8 * cb, d)
        out_ref[...] = seq_ref[...] + e
        embout_ref[...] = e

    return pl.pallas_call(
        body,
        grid=(b, n_tau),
        in_specs=[
            spec_sd,
            pl.BlockSpec(
                (None, 64 * cb, 128), lambda i, t: (i * n_tau + t, 0, 0)
            ),
        ],
        out_specs=[spec_sd, spec_sd],
        out_shape=[
            jax.ShapeDtypeStruct((b, s, d), jnp.float32),
            jax.ShapeDtypeStruct((b, s, d), jnp.float32),
        ],
    )(seq, emb128)


def kernel(seq, coords, seq_id):
    b, s, d = seq.shape
    s_pad = (s + 7) // 8 * 8
    cb = 11
    n_tau = s_pad // (8 * cb)

    ii = jnp.concatenate(
        [coords.astype(jnp.int32), seq_id[..., None].astype(jnp.int32)], axis=-1
    )
    ii = jnp.clip(ii, 0, MAX_LEN - 1)
    ii = jnp.pad(ii, ((0, 0), (1, s_pad - s), (0, 0)))
    idx = ii.reshape(b * s_pad * 4)

    pe = jnp.asarray(_PE)
    emb = _sc_gather(pe, idx, b * s_pad * 4)
    emb128 = emb.reshape(b * n_tau, 64 * cb, 128)
    out, emb_out = _tc_add(seq, emb128, s_pad, cb)
    return (out, emb_out)


# speedup vs baseline: 1.7367x; 1.7367x over previous
import functools
import math

import jax
import jax.numpy as jnp
import numpy as np
from jax import lax
from jax.experimental import pallas as pl
from jax.experimental.pallas import tpu as pltpu
from jax.experimental.pallas import tpu_sc as plsc

D_MODEL = 1024
MAX_LEN = 5000
PE_DIM = D_MODEL // 4


def _pe_table() -> np.ndarray:
    position = np.arange(MAX_LEN, dtype=np.float32)[:, None]
    div_term = np.exp(
        np.arange(0, PE_DIM, 2).astype(np.float32) * (-math.log(10000.0) / PE_DIM)
    )
    pe = np.zeros((MAX_LEN, PE_DIM), dtype=np.float32)
    pe[:, 0::2] = np.sin(position * div_term)
    pe[:, 1::2] = np.cos(position * div_term)
    return pe


_PE = _pe_table()

_CHUNK = 64


def _sc_gather(pe, idx, n_rows):
    info = plsc.get_sparse_core_info()
    nc, ns = info.num_cores, info.num_subcores
    nw = nc * ns
    rows_per_w = n_rows // nw
    assert rows_per_w * nw == n_rows
    n_full = rows_per_w // _CHUNK
    tail = rows_per_w - n_full * _CHUNK
    assert tail % 8 == 0 and rows_per_w % 8 == 0

    mesh = plsc.VectorSubcoreMesh(core_axis_name="c", subcore_axis_name="s")

    scratch = [
        pltpu.VMEM((_CHUNK,), jnp.int32),
        pltpu.VMEM((_CHUNK, PE_DIM), jnp.float32),
        pltpu.SemaphoreType.DMA,
    ]
    if tail:
        scratch += [
            pltpu.VMEM((tail,), jnp.int32),
            pltpu.VMEM((tail, PE_DIM), jnp.float32),
        ]

    @functools.partial(
        pl.kernel,
        mesh=mesh,
        out_type=jax.ShapeDtypeStruct((n_rows, PE_DIM), jnp.float32),
        scratch_types=scratch,
    )
    def k(pe_hbm, idx_hbm, out_hbm, idx_v, rows_v, sg, *tail_refs):
        wid = lax.axis_index("s") * nc + lax.axis_index("c")
        w_base = wid * rows_per_w

        def body(u, carry):
            base = w_base + u * _CHUNK
            pltpu.sync_copy(idx_hbm.at[pl.ds(base, _CHUNK)], idx_v)
            pltpu.async_copy(pe_hbm.at[idx_v], rows_v, sg).wait()
            pltpu.sync_copy(rows_v, out_hbm.at[pl.ds(base, _CHUNK)])
            return carry

        lax.fori_loop(0, n_full, body, 0)

        if tail:
            idx_t, rows_t = tail_refs
            base = w_base + n_full * _CHUNK
            pltpu.sync_copy(idx_hbm.at[pl.ds(base, tail)], idx_t)
            pltpu.async_copy(pe_hbm.at[idx_t], rows_t, sg).wait()
            pltpu.sync_copy(rows_t, out_hbm.at[pl.ds(base, tail)])

    return k(pe, idx)


def _tc_add(seq_t, emb_rows):
    s, b, d = seq_t.shape
    spec_sd = pl.BlockSpec((1, b, d), lambda c: (c, 0, 0))

    def body(seq_ref, emb_ref, out_ref, embout_ref):
        e = emb_ref[...].reshape(1, b, d)
        out_ref[...] = seq_ref[...] + e
        embout_ref[...] = e

    return pl.pallas_call(
        body,
        grid=(s,),
        in_specs=[
            spec_sd,
            pl.BlockSpec((4 * b, PE_DIM), lambda c: (c, 0)),
        ],
        out_specs=[spec_sd, spec_sd],
        out_shape=[
            jax.ShapeDtypeStruct((s, b, d), jnp.float32),
            jax.ShapeDtypeStruct((s, b, d), jnp.float32),
        ],
    )(seq_t, emb_rows)


def kernel(seq, coords, seq_id):
    b, s, d = seq.shape

    ii = jnp.concatenate(
        [coords.astype(jnp.int32), seq_id[..., None].astype(jnp.int32)], axis=-1
    )
    ii = jnp.clip(ii, 0, MAX_LEN - 1)
    ii = jnp.pad(ii, ((0, 0), (1, 0), (0, 0)))
    idx = jnp.transpose(ii, (1, 0, 2)).reshape(s * b * 4)

    pe = jnp.asarray(_PE)
    emb_rows = _sc_gather(pe, idx, s * b * 4)

    seq_t = jnp.transpose(seq, (1, 0, 2))
    out_t, emb_t = _tc_add(seq_t, emb_rows)
    out = jnp.transpose(out_t, (1, 0, 2))
    emb = jnp.transpose(emb_t, (1, 0, 2))
    return (out, emb)


# speedup vs baseline: 1.9798x; 1.1400x over previous
import functools
import math

import jax
import jax.numpy as jnp
import numpy as np
from jax import lax
from jax.experimental import pallas as pl
from jax.experimental.pallas import tpu as pltpu
from jax.experimental.pallas import tpu_sc as plsc

D_MODEL = 1024
MAX_LEN = 5000
PE_DIM = D_MODEL // 4


def _pe_table() -> np.ndarray:
    position = np.arange(MAX_LEN, dtype=np.float32)[:, None]
    div_term = np.exp(
        np.arange(0, PE_DIM, 2).astype(np.float32) * (-math.log(10000.0) / PE_DIM)
    )
    pe = np.zeros((MAX_LEN, PE_DIM), dtype=np.float32)
    pe[:, 0::2] = np.sin(position * div_term)
    pe[:, 1::2] = np.cos(position * div_term)
    return pe


_PE = _pe_table()

_OCHUNK = 64


def _sc_gather(pe, idx, n_out):
    info = plsc.get_sparse_core_info()
    nw = info.num_cores * info.num_subcores
    n_j = n_out // _OCHUNK
    assert n_j * _OCHUNK == n_out
    n_chunks = 4 * n_j

    mesh = plsc.VectorSubcoreMesh(core_axis_name="c", subcore_axis_name="s")

    @functools.partial(
        pl.kernel,
        mesh=mesh,
        out_type=jax.ShapeDtypeStruct((n_out, D_MODEL), jnp.float32),
        scratch_types=[
            pltpu.VMEM((_OCHUNK,), jnp.int32),
            pltpu.VMEM((_OCHUNK, PE_DIM), jnp.float32),
            pltpu.SemaphoreType.DMA,
        ],
    )
    def k(pe_hbm, idx_hbm, out_hbm, idx_v, rows_v, sg):
        wid = lax.axis_index("s") * info.num_cores + lax.axis_index("c")
        n_mine = (n_chunks - wid + nw - 1) // nw

        def body(u, carry):
            chunk = wid + u * nw
            kk = chunk // n_j
            j = chunk % n_j
            pltpu.sync_copy(
                idx_hbm.at[pl.ds(kk * n_out + j * _OCHUNK, _OCHUNK)], idx_v
            )
            pltpu.async_copy(pe_hbm.at[idx_v], rows_v, sg).wait()
            pltpu.sync_copy(
                rows_v,
                out_hbm.at[pl.ds(j * _OCHUNK, _OCHUNK), pl.ds(kk * PE_DIM, PE_DIM)],
            )
            return carry

        lax.fori_loop(0, n_mine, body, 0)

    return k(pe, idx)


def _tc_add(seq_t, emb3):
    s, b, d = seq_t.shape
    spec = pl.BlockSpec((1, b, d), lambda c: (c, 0, 0))

    def body(seq_ref, emb_ref, out_ref):
        out_ref[...] = seq_ref[...] + emb_ref[...]

    return pl.pallas_call(
        body,
        grid=(s,),
        in_specs=[spec, spec],
        out_specs=spec,
        out_shape=jax.ShapeDtypeStruct((s, b, d), jnp.float32),
    )(seq_t, emb3)


def kernel(seq, coords, seq_id):
    b, s, d = seq.shape

    ii = jnp.concatenate(
        [coords.astype(jnp.int32), seq_id[..., None].astype(jnp.int32)], axis=-1
    )
    ii = jnp.clip(ii, 0, MAX_LEN - 1)
    ii = jnp.pad(ii, ((0, 0), (1, 0), (0, 0)))
    idx = jnp.transpose(ii, (2, 1, 0)).reshape(4 * s * b)

    pe = jnp.asarray(_PE)
    emb2 = _sc_gather(pe, idx, s * b)
    emb3 = emb2.reshape(s, b, d)

    seq_t = jnp.transpose(seq, (1, 0, 2))
    out_t = _tc_add(seq_t, emb3)
    out = jnp.transpose(out_t, (1, 0, 2))
    emb = jnp.transpose(emb3, (1, 0, 2))
    return (out, emb)


# speedup vs baseline: 2.0303x; 1.0255x over previous
import functools
import math

import jax
import jax.numpy as jnp
import numpy as np
from jax import lax
from jax.experimental import pallas as pl
from jax.experimental.pallas import tpu as pltpu
from jax.experimental.pallas import tpu_sc as plsc

D_MODEL = 1024
MAX_LEN = 5000
PE_DIM = D_MODEL // 4


def _pe_table() -> np.ndarray:
    position = np.arange(MAX_LEN, dtype=np.float32)[:, None]
    div_term = np.exp(
        np.arange(0, PE_DIM, 2).astype(np.float32) * (-math.log(10000.0) / PE_DIM)
    )
    pe = np.zeros((MAX_LEN, PE_DIM), dtype=np.float32)
    pe[:, 0::2] = np.sin(position * div_term)
    pe[:, 1::2] = np.cos(position * div_term)
    return pe


_PE = _pe_table()

_OCHUNK = 64


def _sc_gather(pe, idx, n_out):
    info = plsc.get_sparse_core_info()
    nw = info.num_cores * info.num_subcores
    n_j = n_out // _OCHUNK
    assert n_j * _OCHUNK == n_out
    n_chunks = 4 * n_j

    mesh = plsc.VectorSubcoreMesh(core_axis_name="c", subcore_axis_name="s")

    @functools.partial(
        pl.kernel,
        mesh=mesh,
        out_type=jax.ShapeDtypeStruct((n_out, D_MODEL), jnp.float32),
        scratch_types=[
            pltpu.VMEM((_OCHUNK,), jnp.int32),
            pltpu.VMEM((_OCHUNK, PE_DIM), jnp.float32),
            pltpu.SemaphoreType.DMA,
        ],
    )
    def k(pe_hbm, idx_hbm, out_hbm, idx_v, rows_v, sg):
        wid = lax.axis_index("s") * info.num_cores + lax.axis_index("c")
        n_mine = (n_chunks - wid + nw - 1) // nw

        def body(u, carry):
            chunk = wid + u * nw
            kk = chunk // n_j
            j = chunk % n_j
            pltpu.sync_copy(
                idx_hbm.at[pl.ds(kk * n_out + j * _OCHUNK, _OCHUNK)], idx_v
            )
            pltpu.async_copy(pe_hbm.at[idx_v], rows_v, sg).wait()
            pltpu.sync_copy(
                rows_v,
                out_hbm.at[pl.ds(j * _OCHUNK, _OCHUNK), pl.ds(kk * PE_DIM, PE_DIM)],
            )
            return carry

        lax.fori_loop(0, n_mine, body, 0)

    return k(pe, idx)


def _tc_add_group(seq_t, emb3_g, c0, accs):
    s, b, d = seq_t.shape
    sg = emb3_g.shape[0]
    acc_spec = pl.BlockSpec((1, b, d), lambda c: (c + c0, 0, 0))
    emb_spec = pl.BlockSpec((1, b, d), lambda c: (c, 0, 0))
    any_spec = pl.BlockSpec(memory_space=pl.ANY)

    def body(seq_ref, emb_ref, *rest):
        out_ref, embout_ref = rest[-2:]
        e = emb_ref[...]
        out_ref[...] = seq_ref[...] + e
        embout_ref[...] = e

    operands = [seq_t, emb3_g]
    in_specs = [acc_spec, emb_spec]
    aliases = {}
    if accs is not None:
        operands += list(accs)
        in_specs += [any_spec, any_spec]
        aliases = {2: 0, 3: 1}

    return pl.pallas_call(
        body,
        grid=(sg,),
        in_specs=in_specs,
        out_specs=[acc_spec, acc_spec],
        out_shape=[
            jax.ShapeDtypeStruct((s, b, d), jnp.float32),
            jax.ShapeDtypeStruct((s, b, d), jnp.float32),
        ],
        input_output_aliases=aliases,
    )(*operands)


_N_GROUPS = 4


def kernel(seq, coords, seq_id):
    b, s, d = seq.shape

    ii = jnp.concatenate(
        [coords.astype(jnp.int32), seq_id[..., None].astype(jnp.int32)], axis=-1
    )
    ii = jnp.clip(ii, 0, MAX_LEN - 1)
    ii = jnp.pad(ii, ((0, 0), (1, 0), (0, 0)))
    ii_t = jnp.transpose(ii, (2, 1, 0))

    pe = jnp.asarray(_PE)
    seq_t = jnp.transpose(seq, (1, 0, 2))

    base = s // _N_GROUPS
    bounds = [g * base for g in range(_N_GROUPS)] + [s]
    accs = None
    for g in range(_N_GROUPS):
        c0, c1 = bounds[g], bounds[g + 1]
        sg = c1 - c0
        idx_g = ii_t[:, c0:c1, :].reshape(4 * sg * b)
        emb_g = _sc_gather(pe, idx_g, sg * b)
        accs = _tc_add_group(seq_t, emb_g.reshape(sg, b, d), c0, accs)

    out = jnp.transpose(accs[0], (1, 0, 2))
    emb = jnp.transpose(accs[1], (1, 0, 2))
    return (out, emb)


# speedup vs baseline: 2.0376x; 1.0036x over previous
import functools
import math

import jax
import jax.numpy as jnp
import numpy as np
from jax import lax
from jax.experimental import pallas as pl
from jax.experimental.pallas import tpu as pltpu
from jax.experimental.pallas import tpu_sc as plsc

D_MODEL = 1024
MAX_LEN = 5000
PE_DIM = D_MODEL // 4


def _pe_table() -> np.ndarray:
    position = np.arange(MAX_LEN, dtype=np.float32)[:, None]
    div_term = np.exp(
        np.arange(0, PE_DIM, 2).astype(np.float32) * (-math.log(10000.0) / PE_DIM)
    )
    pe = np.zeros((MAX_LEN, PE_DIM), dtype=np.float32)
    pe[:, 0::2] = np.sin(position * div_term)
    pe[:, 1::2] = np.cos(position * div_term)
    return pe


_PE = _pe_table()

_OCHUNK = 64


def _sc_gather(pe, idx, n_out):
    info = plsc.get_sparse_core_info()
    nw = info.num_cores * info.num_subcores
    n_j = n_out // _OCHUNK
    assert n_j * _OCHUNK == n_out
    n_chunks = 4 * n_j

    mesh = plsc.VectorSubcoreMesh(core_axis_name="c", subcore_axis_name="s")

    @functools.partial(
        pl.kernel,
        mesh=mesh,
        out_type=jax.ShapeDtypeStruct((n_out, D_MODEL), jnp.float32),
        scratch_types=[
            pltpu.VMEM((_OCHUNK,), jnp.int32),
            pltpu.VMEM((_OCHUNK,), jnp.int32),
            pltpu.VMEM((_OCHUNK, PE_DIM), jnp.float32),
            pltpu.VMEM((_OCHUNK, PE_DIM), jnp.float32),
            pltpu.SemaphoreType.DMA,
            pltpu.SemaphoreType.DMA,
        ],
    )
    def k(pe_hbm, idx_hbm, out_hbm, idx_v0, idx_v1, rows_v0, rows_v1, sg0, sg1):
        wid = lax.axis_index("s") * info.num_cores + lax.axis_index("c")
        n_mine = (n_chunks - wid + nw - 1) // nw
        idx_vs = (idx_v0, idx_v1)
        rows_vs = (rows_v0, rows_v1)
        sgs = (sg0, sg1)

        def start(buf, u):
            @pl.when(u < n_mine)
            def _():
                chunk = wid + u * nw
                kk = chunk // n_j
                j = chunk % n_j
                pltpu.sync_copy(
                    idx_hbm.at[pl.ds(kk * n_out + j * _OCHUNK, _OCHUNK)], idx_vs[buf]
                )
                pltpu.async_copy(pe_hbm.at[idx_vs[buf]], rows_vs[buf], sgs[buf])

        def drain(buf, u):
            @pl.when(u < n_mine)
            def _():
                chunk = wid + u * nw
                kk = chunk // n_j
                j = chunk % n_j
                pltpu.make_async_copy(
                    pe_hbm.at[idx_vs[buf]], rows_vs[buf], sgs[buf]
                ).wait()
                pltpu.sync_copy(
                    rows_vs[buf],
                    out_hbm.at[
                        pl.ds(j * _OCHUNK, _OCHUNK), pl.ds(kk * PE_DIM, PE_DIM)
                    ],
                )

        start(0, 0)

        def body(t, carry):
            u0 = 2 * t
            start(1, u0 + 1)
            drain(0, u0)
            start(0, u0 + 2)
            drain(1, u0 + 1)
            return carry

        n_pairs = ((n_chunks + nw - 1) // nw + 1) // 2
        lax.fori_loop(0, n_pairs, body, 0)

    return k(pe, idx)


def _tc_add_group(seq_t, emb3_g, c0, accs):
    s, b, d = seq_t.shape
    sg = emb3_g.shape[0]
    acc_spec = pl.BlockSpec((1, b, d), lambda c: (c + c0, 0, 0))
    emb_spec = pl.BlockSpec((1, b, d), lambda c: (c, 0, 0))
    any_spec = pl.BlockSpec(memory_space=pl.ANY)

    def body(seq_ref, emb_ref, *rest):
        out_ref, embout_ref = rest[-2:]
        e = emb_ref[...]
        out_ref[...] = seq_ref[...] + e
        embout_ref[...] = e

    operands = [seq_t, emb3_g]
    in_specs = [acc_spec, emb_spec]
    aliases = {}
    if accs is not None:
        operands += list(accs)
        in_specs += [any_spec, any_spec]
        aliases = {2: 0, 3: 1}

    return pl.pallas_call(
        body,
        grid=(sg,),
        in_specs=in_specs,
        out_specs=[acc_spec, acc_spec],
        out_shape=[
            jax.ShapeDtypeStruct((s, b, d), jnp.float32),
            jax.ShapeDtypeStruct((s, b, d), jnp.float32),
        ],
        input_output_aliases=aliases,
    )(*operands)


_N_GROUPS = 4


def kernel(seq, coords, seq_id):
    b, s, d = seq.shape

    ii = jnp.concatenate(
        [coords.astype(jnp.int32), seq_id[..., None].astype(jnp.int32)], axis=-1
    )
    ii = jnp.clip(ii, 0, MAX_LEN - 1)
    ii = jnp.pad(ii, ((0, 0), (1, 0), (0, 0)))
    ii_t = jnp.transpose(ii, (2, 1, 0))

    pe = jnp.asarray(_PE)
    seq_t = jnp.transpose(seq, (1, 0, 2))

    base = s // _N_GROUPS
    bounds = [g * base for g in range(_N_GROUPS)] + [s]
    accs = None
    for g in range(_N_GROUPS):
        c0, c1 = bounds[g], bounds[g + 1]
        sg = c1 - c0
        idx_g = ii_t[:, c0:c1, :].reshape(4 * sg * b)
        emb_g = _sc_gather(pe, idx_g, sg * b)
        accs = _tc_add_group(seq_t, emb_g.reshape(sg, b, d), c0, accs)

    out = jnp.transpose(accs[0], (1, 0, 2))
    emb = jnp.transpose(accs[1], (1, 0, 2))
    return (out, emb)
